# baseline (speedup 1.0000x reference)
import jax
import jax.numpy as jnp
from jax import lax
from jax.experimental import pallas as pl
from jax.experimental.pallas import tpu as pltpu
from jax.experimental.pallas import tpu_sc as plsc

N = 100000
E = 6400000
CH = 10
ED = 4
ACCEL_SCALE = 0.005
MAX_VEL = 0.02

CHUNK = 6272
NPAD = 16 * CHUNK
K = 1024
SUB = 2 * K // 128
NBLK = E // K
NWORK = 32
FULL, EXTRA = divmod(NBLK, NWORK)


def _proj_body(x_ref, w2_ref, b2_ref, p_ref):
    p_ref[...] = (
        jnp.dot(x_ref[...], w2_ref[...], preferred_element_type=jnp.float32)
        + b2_ref[...]
    )


_ROW_B = 8192
_ROW_GRID = (N + _ROW_B - 1) // _ROW_B


def _project(x, w2, b2):
    return pl.pallas_call(
        _proj_body,
        grid=(_ROW_GRID,),
        in_specs=[
            pl.BlockSpec((_ROW_B, CH), lambda i: (i, 0)),
            pl.BlockSpec((CH, 2), lambda i: (0, 0)),
            pl.BlockSpec((1, 2), lambda i: (0, 0)),
        ],
        out_specs=pl.BlockSpec((_ROW_B, 2), lambda i: (i, 0)),
        out_shape=jax.ShapeDtypeStruct((N, 2), jnp.float32),
    )(x, w2, b2)


_ER = E // 32


def _eproj_body(a_ref, s_ref, q_ref):
    q_ref[...] = jnp.dot(a_ref[...], s_ref[...],
                         preferred_element_type=jnp.float32)


_EB = 2000


def _eproject(ea2, smat):
    return pl.pallas_call(
        _eproj_body,
        grid=(_ER // _EB,),
        in_specs=[
            pl.BlockSpec((_EB, 128), lambda i: (i, 0)),
            pl.BlockSpec((128, 128), lambda i: (0, 0)),
        ],
        out_specs=pl.BlockSpec((_EB, 128), lambda i: (i, 0)),
        out_shape=jax.ShapeDtypeStruct((_ER, 128), jnp.float32),
    )(ea2, smat)


def _build_smat(wa2):
    li = jnp.arange(128)[:, None]
    sj = jnp.arange(128)[None, :]
    w0 = wa2[li % 4, 0]
    w1 = wa2[li % 4, 1]
    sel0 = (li // 4 == sj) & (sj < 32)
    sel1 = (li // 4 == sj - 32) & (sj >= 32) & (sj < 64)
    return jnp.where(sel0, w0, 0.0) + jnp.where(sel1, w1, 0.0)


def _sc_edges_body(ptab_hbm, ei_hbm, q_hbm, zeros_hbm,
                   out_hbm, ptab_v, src_v, dst_v, didx_v, q_v, msg_v,
                   acc_sh):
    ci = lax.axis_index("c")
    si = lax.axis_index("s")
    wid = si * 2 + ci
    iota = lax.broadcasted_iota(jnp.int32, (16,), 0)
    zsplat = iota * 0

    base = si * 2 * CHUNK
    pltpu.sync_copy(zeros_hbm, acc_sh.at[pl.ds(base, 2 * CHUNK)])

    pltpu.sync_copy(ptab_hbm, ptab_v)
    plsc.subcore_barrier()

    nblk = FULL + jnp.where(wid < EXTRA, 1, 0)

    def _blk_body(jb, c):
        b = wid + jb * NWORK
        ebase = b * K
        pltpu.sync_copy(ei_hbm.at[0, pl.ds(ebase, K)], src_v)
        pltpu.sync_copy(ei_hbm.at[1, pl.ds(ebase, K)], dst_v)
        pltpu.sync_copy(q_hbm.at[pl.ds(b * (K // 32), K // 32)], q_v)

        def _step(i, cc):
            eoff = i * 16
            lane_e = eoff + iota
            src16 = src_v[pl.ds(eoff, 16)]
            dst16 = dst_v[pl.ds(eoff, 16)]
            w = plsc.load_gather(ptab_v, [src16])
            p0 = plsc.bitcast(w << 16, jnp.float32)
            p1 = plsc.bitcast(w & jnp.int32(-65536), jnp.float32)
            qrow = zsplat + (eoff >> 5)
            qcol = (eoff & 31) + iota
            z0 = p0 + plsc.load_gather(q_v, [qrow, qcol])
            z1 = p1 + plsc.load_gather(q_v, [qrow, qcol + 32])
            t0 = 1.0 - 2.0 / (jnp.exp(z0 + z0) + 1.0)
            t1 = 1.0 - 2.0 / (jnp.exp(z1 + z1) + 1.0)
            m_base = lane_e * 2
            plsc.store_scatter(msg_v, [m_base], t0)
            plsc.store_scatter(msg_v, [m_base + 1], t1)
            d2 = dst16 + dst16
            plsc.store_scatter(didx_v, [m_base >> 7, m_base & 127], d2)
            m1 = m_base + 1
            plsc.store_scatter(didx_v, [m1 >> 7, m1 & 127], d2 + 1)
            return cc

        lax.fori_loop(0, K // 16, _step, 0)
        for j in range(SUB):
            pltpu.sync_copy(msg_v.at[pl.ds(j * 128, 128)],
                            acc_sh.at[didx_v.at[j]], add=True)
        return c

    lax.fori_loop(0, nblk, _blk_body, 0)
    plsc.subcore_barrier()
    pltpu.sync_copy(acc_sh.at[pl.ds(base, 2 * CHUNK)],
                    out_hbm.at[ci, pl.ds(base, 2 * CHUNK)])


_sc_edges = pl.kernel(
    _sc_edges_body,
    out_type=jax.ShapeDtypeStruct((2, 2 * NPAD), jnp.float32),
    mesh=plsc.VectorSubcoreMesh(core_axis_name="c", subcore_axis_name="s"),
    compiler_params=pltpu.CompilerParams(needs_layout_passes=False),
    scratch_types=[
        pltpu.VMEM((N,), jnp.int32),
        pltpu.VMEM((K,), jnp.int32),
        pltpu.VMEM((K,), jnp.int32),
        pltpu.VMEM((SUB, 128), jnp.int32),
        pltpu.VMEM((K // 32, 128), jnp.float32),
        pltpu.VMEM((2 * K,), jnp.float32),
        pltpu.VMEM_SHARED((2 * NPAD,), jnp.float32),
    ],
)


def _update_body(x_ref, h0_ref, h1_ref, o_ref):
    x = x_ref[...]
    h2 = h0_ref[...] + h1_ref[...]
    cm = x[:, 4:5] > 0.5
    cmf = cm.astype(jnp.float32)
    acc = h2 * cmf * ACCEL_SCALE
    vel = jnp.clip(x[:, 2:4] + acc, -MAX_VEL, MAX_VEL) * cmf
    pos = jnp.remainder(x[:, 0:2] + vel + 1.0, 2.0) - 1.0
    pos = jnp.where(cm, pos, x[:, 0:2])
    o_ref[...] = jnp.concatenate([pos, vel, x[:, 4:]], axis=1)


def _update(x, h0, h1):
    return pl.pallas_call(
        _update_body,
        grid=(_ROW_GRID,),
        in_specs=[
            pl.BlockSpec((_ROW_B, CH), lambda i: (i, 0)),
            pl.BlockSpec((_ROW_B, 2), lambda i: (i, 0)),
            pl.BlockSpec((_ROW_B, 2), lambda i: (i, 0)),
        ],
        out_specs=pl.BlockSpec((_ROW_B, CH), lambda i: (i, 0)),
        out_shape=jax.ShapeDtypeStruct((N, CH), jnp.float32),
    )(x, h0, h1)


def kernel(x, edge_index, edge_attr, W, b):
    p = _project(x, W[:CH, :2], b[:2].reshape(1, 2))
    ppack = lax.bitcast_convert_type(p.astype(jnp.bfloat16), jnp.int32)
    qrows = _eproject(edge_attr.reshape(_ER, 128), _build_smat(W[CH:, :2]))
    zeros = jnp.zeros((2 * CHUNK,), jnp.float32)
    hacc = _sc_edges(ppack, edge_index, qrows, zeros)
    h0 = hacc[0].reshape(NPAD, 2)[:N]
    h1 = hacc[1].reshape(NPAD, 2)[:N]
    return _update(x, h0, h1)


# speedup vs baseline: 5.7402x; 5.7402x over previous
import jax
import jax.numpy as jnp
from jax import lax
from jax.experimental import pallas as pl
from jax.experimental.pallas import tpu as pltpu
from jax.experimental.pallas import tpu_sc as plsc

N = 100000
E = 6400000
CH = 10
ED = 4
ACCEL_SCALE = 0.005
MAX_VEL = 0.02

CHUNK = 6272
NPAD = 16 * CHUNK
K = 1024
SUB = 2 * K // 128
NBLK = E // K
NWORK = 32
FULL, EXTRA = divmod(NBLK, NWORK)


def _proj_body(x_ref, w2_ref, b2_ref, p_ref):
    p_ref[...] = (
        jnp.dot(x_ref[...], w2_ref[...], preferred_element_type=jnp.float32)
        + b2_ref[...]
    )


_ROW_B = 8192
_ROW_GRID = (N + _ROW_B - 1) // _ROW_B


def _project(x, w2, b2):
    return pl.pallas_call(
        _proj_body,
        grid=(_ROW_GRID,),
        in_specs=[
            pl.BlockSpec((_ROW_B, CH), lambda i: (i, 0)),
            pl.BlockSpec((CH, 2), lambda i: (0, 0)),
            pl.BlockSpec((1, 2), lambda i: (0, 0)),
        ],
        out_specs=pl.BlockSpec((_ROW_B, 2), lambda i: (i, 0)),
        out_shape=jax.ShapeDtypeStruct((N, 2), jnp.float32),
    )(x, w2, b2)


_ER = E // 32


def _eproj_body(a_ref, s_ref, q_ref):
    q_ref[...] = jnp.dot(a_ref[...], s_ref[...],
                         preferred_element_type=jnp.float32)


_EB = 2000


def _eproject(ea2, smat):
    return pl.pallas_call(
        _eproj_body,
        grid=(_ER // _EB,),
        in_specs=[
            pl.BlockSpec((_EB, 128), lambda i: (i, 0)),
            pl.BlockSpec((128, 128), lambda i: (0, 0)),
        ],
        out_specs=pl.BlockSpec((_EB, 128), lambda i: (i, 0)),
        out_shape=jax.ShapeDtypeStruct((_ER, 128), jnp.float32),
    )(ea2, smat)


def _build_smat(wa2):
    li = jnp.arange(128)[:, None]
    sj = jnp.arange(128)[None, :]
    w0 = wa2[li % 4, 0]
    w1 = wa2[li % 4, 1]
    sel0 = (li // 4 == sj) & (sj < 32)
    sel1 = (li // 4 == sj - 32) & (sj >= 32) & (sj < 64)
    return jnp.where(sel0, w0, 0.0) + jnp.where(sel1, w1, 0.0)


def _sc_edges_body(ptab_hbm, ei_hbm, attr_hbm, w_hbm, zeros_hbm,
                   out_hbm, ptab_v, src_v, dst_v, didx_v, a0_v, a1_v, a2_v,
                   a3_v, wbuf_v, msg_v, acc_sh):
    ci = lax.axis_index("c")
    si = lax.axis_index("s")
    wid = si * 2 + ci
    iota = lax.broadcasted_iota(jnp.int32, (16,), 0)
    zsplat = iota * 0

    base = si * 2 * CHUNK
    pltpu.sync_copy(zeros_hbm, acc_sh.at[pl.ds(base, 2 * CHUNK)])

    pltpu.sync_copy(ptab_hbm, ptab_v)
    pltpu.sync_copy(w_hbm, wbuf_v)
    plsc.subcore_barrier()

    wa = [[plsc.load_gather(wbuf_v, [zsplat + (c * 2 + j)])
           for j in (0, 1)] for c in range(ED)]

    nblk = FULL + jnp.where(wid < EXTRA, 1, 0)

    def _blk_body(jb, c):
        b = wid + jb * NWORK
        ebase = b * K
        pltpu.sync_copy(ei_hbm.at[0, pl.ds(ebase, K)], src_v)
        pltpu.sync_copy(ei_hbm.at[1, pl.ds(ebase, K)], dst_v)
        pltpu.sync_copy(attr_hbm.at[0, pl.ds(ebase, K)], a0_v)
        pltpu.sync_copy(attr_hbm.at[1, pl.ds(ebase, K)], a1_v)
        pltpu.sync_copy(attr_hbm.at[2, pl.ds(ebase, K)], a2_v)
        pltpu.sync_copy(attr_hbm.at[3, pl.ds(ebase, K)], a3_v)

        def _step(i, cc):
            eoff = i * 16
            lane_e = eoff + iota
            src16 = src_v[pl.ds(eoff, 16)]
            dst16 = dst_v[pl.ds(eoff, 16)]
            w = plsc.load_gather(ptab_v, [src16])
            p0 = plsc.bitcast(w << 16, jnp.float32)
            p1 = plsc.bitcast(w & jnp.int32(-65536), jnp.float32)
            a0 = a0_v[pl.ds(eoff, 16)]
            a1 = a1_v[pl.ds(eoff, 16)]
            a2 = a2_v[pl.ds(eoff, 16)]
            a3 = a3_v[pl.ds(eoff, 16)]
            z0 = p0 + a0 * wa[0][0] + a1 * wa[1][0] + a2 * wa[2][0] + a3 * wa[3][0]
            z1 = p1 + a0 * wa[0][1] + a1 * wa[1][1] + a2 * wa[2][1] + a3 * wa[3][1]
            t0 = 1.0 - 2.0 / (jnp.exp(z0 + z0) + 1.0)
            t1 = 1.0 - 2.0 / (jnp.exp(z1 + z1) + 1.0)
            m_base = lane_e * 2
            plsc.store_scatter(msg_v, [m_base], t0)
            plsc.store_scatter(msg_v, [m_base + 1], t1)
            d2 = dst16 + dst16
            plsc.store_scatter(didx_v, [m_base >> 7, m_base & 127], d2)
            m1 = m_base + 1
            plsc.store_scatter(didx_v, [m1 >> 7, m1 & 127], d2 + 1)
            return cc

        lax.fori_loop(0, K // 16, _step, 0)
        for j in range(SUB):
            pltpu.sync_copy(msg_v.at[pl.ds(j * 128, 128)],
                            acc_sh.at[didx_v.at[j]], add=True)
        return c

    lax.fori_loop(0, nblk, _blk_body, 0)
    plsc.subcore_barrier()
    pltpu.sync_copy(acc_sh.at[pl.ds(base, 2 * CHUNK)],
                    out_hbm.at[ci, pl.ds(base, 2 * CHUNK)])


_sc_edges = pl.kernel(
    _sc_edges_body,
    out_type=jax.ShapeDtypeStruct((2, 2 * NPAD), jnp.float32),
    mesh=plsc.VectorSubcoreMesh(core_axis_name="c", subcore_axis_name="s"),
    compiler_params=pltpu.CompilerParams(needs_layout_passes=False),
    scratch_types=[
        pltpu.VMEM((N,), jnp.int32),
        pltpu.VMEM((K,), jnp.int32),
        pltpu.VMEM((K,), jnp.int32),
        pltpu.VMEM((SUB, 128), jnp.int32),
        pltpu.VMEM((K,), jnp.float32),
        pltpu.VMEM((K,), jnp.float32),
        pltpu.VMEM((K,), jnp.float32),
        pltpu.VMEM((K,), jnp.float32),
        pltpu.VMEM((128,), jnp.float32),
        pltpu.VMEM((2 * K,), jnp.float32),
        pltpu.VMEM_SHARED((2 * NPAD,), jnp.float32),
    ],
)


def _update_body(x_ref, h0_ref, h1_ref, o_ref):
    x = x_ref[...]
    h2 = h0_ref[...] + h1_ref[...]
    cm = x[:, 4:5] > 0.5
    cmf = cm.astype(jnp.float32)
    acc = h2 * cmf * ACCEL_SCALE
    vel = jnp.clip(x[:, 2:4] + acc, -MAX_VEL, MAX_VEL) * cmf
    pos = jnp.remainder(x[:, 0:2] + vel + 1.0, 2.0) - 1.0
    pos = jnp.where(cm, pos, x[:, 0:2])
    o_ref[...] = jnp.concatenate([pos, vel, x[:, 4:]], axis=1)


def _update(x, h0, h1):
    return pl.pallas_call(
        _update_body,
        grid=(_ROW_GRID,),
        in_specs=[
            pl.BlockSpec((_ROW_B, CH), lambda i: (i, 0)),
            pl.BlockSpec((_ROW_B, 2), lambda i: (i, 0)),
            pl.BlockSpec((_ROW_B, 2), lambda i: (i, 0)),
        ],
        out_specs=pl.BlockSpec((_ROW_B, CH), lambda i: (i, 0)),
        out_shape=jax.ShapeDtypeStruct((N, CH), jnp.float32),
    )(x, h0, h1)


def kernel(x, edge_index, edge_attr, W, b):
    p = _project(x, W[:CH, :2], b[:2].reshape(1, 2))
    ppack = lax.bitcast_convert_type(p.astype(jnp.bfloat16), jnp.int32)
    wflat = jnp.pad(W[CH:, :2].reshape(2 * ED), (0, 128 - 2 * ED))
    zeros = jnp.zeros((2 * CHUNK,), jnp.float32)
    hacc = _sc_edges(ppack, edge_index, edge_attr.T, wflat, zeros)
    h0 = hacc[0].reshape(NPAD, 2)[:N]
    h1 = hacc[1].reshape(NPAD, 2)[:N]
    return _update(x, h0, h1)
